# Initial kernel scaffold; baseline (speedup 1.0000x reference)
import jax
import jax.numpy as jnp
from jax.experimental import pallas as pl


def kernel(x, tproj_W, tproj_b, enc_emb_W, enc_emb_b, enc_W1, enc_b1, enc_W2, enc_b2, enc_fc_W, enc_fc_b, dec_W1, dec_b1, dec_W2, dec_b2, dec_fc_W, dec_fc_b, edge_index):
    raise NotImplementedError("write your pallas kernel here")


# speedup vs baseline: 1.1155x; 1.1155x over previous
import functools

import jax
import jax.numpy as jnp
import numpy as np
from jax.experimental import pallas as pl

_N, _D, _H = 10000, 128, 128


def _mm_kernel(x_ref, w_ref, b_ref, o_ref):
    o_ref[...] = (
        jnp.dot(x_ref[...], w_ref[...], preferred_element_type=jnp.float32)
        + b_ref[...]
    )


def _mm(x, w, b, block=512):
    m, k = x.shape
    _, n = w.shape
    pad = (-m) % block
    if pad:
        x = jnp.pad(x, ((0, pad), (0, 0)))
    mp = m + pad
    out = pl.pallas_call(
        _mm_kernel,
        grid=(mp // block,),
        in_specs=[
            pl.BlockSpec((block, k), lambda i: (i, 0)),
            pl.BlockSpec((k, n), lambda i: (0, 0)),
            pl.BlockSpec((1, n), lambda i: (0, 0)),
        ],
        out_specs=pl.BlockSpec((block, n), lambda i: (i, 0)),
        out_shape=jax.ShapeDtypeStruct((mp, n), jnp.float32),
    )(x, w, b.reshape(1, n))
    return out[:m] if pad else out


def _edge_conv(h, W1, b1, W2, b2, src, dst):
    W1a, W1b = W1[:_H], W1[_H:]
    A = _mm(h, W1a - W1b, b1)
    B = _mm(h, W1b, jnp.zeros((_H,), jnp.float32))
    out0 = _mm(jnp.maximum(A + B, 0.0), W2, b2)
    S = jnp.maximum(A[dst] + B[src], 0.0)
    M = _mm(S, W2, b2)
    seg = jax.ops.segment_max(M, dst, num_segments=_N)
    return jnp.maximum(out0, seg)


def kernel(x, tproj_W, tproj_b, enc_emb_W, enc_emb_b, enc_W1, enc_b1,
           enc_W2, enc_b2, enc_fc_W, enc_fc_b, dec_W1, dec_b1, dec_W2,
           dec_b2, dec_fc_W, dec_fc_b, edge_index):
    n = x.shape[0]
    timesteps = 1000
    s = 0.008
    steps = jnp.linspace(0.0, float(timesteps), timesteps + 1)
    alpha_bar = jnp.cos((steps / timesteps + s) / (1 + s) * jnp.pi / 2) ** 2
    rk = jax.random.key(42)
    k1, k2 = jax.random.split(rk)
    t = jax.random.randint(k1, (1,), 0, timesteps)
    noise = jax.random.normal(k2, x.shape, dtype=x.dtype)
    ab_t = alpha_bar[t][:, None]
    x_noisy = jnp.sqrt(ab_t) * x + jnp.sqrt(1.0 - ab_t) * noise
    tf = t.astype(jnp.float32) / timesteps
    freq = jnp.exp(jnp.linspace(-4.0, 4.0, 32))
    emb = jnp.concatenate([jnp.sin(tf * freq), jnp.cos(tf * freq)], axis=-1)
    t_emb = emb @ tproj_W + tproj_b
    h = x_noisy + t_emb[None, :]

    src, dst = edge_index[0], edge_index[1]

    h = _mm(h, enc_emb_W, enc_emb_b)
    for i in range(4):
        h = _edge_conv(h, enc_W1[i], enc_b1[i], enc_W2[i], enc_b2[i], src, dst)
    h = _mm(h, enc_fc_W, enc_fc_b)
    for i in range(4):
        h = _edge_conv(h, dec_W1[i], dec_b1[i], dec_W2[i], dec_b2[i], src, dst)
    score = _mm(h, dec_fc_W, dec_fc_b)
    loss = jnp.mean((score - noise) ** 2)
    return loss


# speedup vs baseline: 1.3367x; 1.1983x over previous
import functools

import jax
import jax.numpy as jnp
import numpy as np
from jax import lax
from jax.experimental import pallas as pl
from jax.experimental.pallas import tpu as pltpu

_N, _H = 10000, 128
_NB = 256
_EC = 1024
_NBLK = 40
_NPAD = _NB * _NBLK
_E = 320000
_NCHUNK = _E // _EC + _NBLK
_EP = _NCHUNK * _EC


def _mm_kernel(x_ref, w_ref, b_ref, o_ref):
    o_ref[...] = (
        jnp.dot(x_ref[...], w_ref[...], preferred_element_type=jnp.float32)
        + b_ref[...]
    )


def _mm(x, w, b, block=512):
    m, k = x.shape
    _, n = w.shape
    pad = (-m) % block
    if pad:
        x = jnp.pad(x, ((0, pad), (0, 0)))
    mp = m + pad
    out = pl.pallas_call(
        _mm_kernel,
        grid=(mp // block,),
        in_specs=[
            pl.BlockSpec((block, k), lambda i: (i, 0)),
            pl.BlockSpec((k, n), lambda i: (0, 0)),
            pl.BlockSpec((1, n), lambda i: (0, 0)),
        ],
        out_specs=pl.BlockSpec((block, n), lambda i: (i, 0)),
        out_shape=jax.ShapeDtypeStruct((mp, n), jnp.float32),
    )(x, w, b.reshape(1, n))
    return out[:m] if pad else out


def _dense_kernel(h_ref, wd_ref, wb_ref, w2_ref, b1_ref, b2_ref,
                  oa_ref, ob_ref, oo_ref):
    h = h_ref[...]
    a = jnp.dot(h, wd_ref[...], preferred_element_type=jnp.float32) + b1_ref[...]
    bm = jnp.dot(h, wb_ref[...], preferred_element_type=jnp.float32)
    oa_ref[...] = a
    ob_ref[...] = bm
    oo_ref[...] = (
        jnp.dot(jnp.maximum(a + bm, 0.0), w2_ref[...],
                preferred_element_type=jnp.float32)
        + b2_ref[...]
    )


def _dense_phase(h, wd, wb, w2, b1, b2, *, npad, block=512):
    nblk = npad // block
    hd = h.shape[1]
    sd = jax.ShapeDtypeStruct((npad, hd), jnp.float32)
    return pl.pallas_call(
        _dense_kernel,
        grid=(nblk,),
        in_specs=[
            pl.BlockSpec((block, hd), lambda i: (i, 0)),
            pl.BlockSpec((hd, hd), lambda i: (0, 0)),
            pl.BlockSpec((hd, hd), lambda i: (0, 0)),
            pl.BlockSpec((hd, hd), lambda i: (0, 0)),
            pl.BlockSpec((1, hd), lambda i: (0, 0)),
            pl.BlockSpec((1, hd), lambda i: (0, 0)),
        ],
        out_specs=[
            pl.BlockSpec((block, hd), lambda i: (i, 0)),
            pl.BlockSpec((block, hd), lambda i: (i, 0)),
            pl.BlockSpec((block, hd), lambda i: (i, 0)),
        ],
        out_shape=[sd, sd, sd],
    )(h, wd, wb, w2, b1.reshape(1, hd), b2.reshape(1, hd))


def _setup_edges(src, dst, *, nb, ec, nblk, ep, e):
    nchunk = ep // ec
    dst_s, src_s = lax.sort_key_val(dst, src)
    bounds = (jnp.arange(nblk, dtype=jnp.int32) + 1) * nb
    hi = jnp.searchsorted(dst_s, bounds, side="left").astype(jnp.int32)
    start = jnp.concatenate([jnp.zeros((1,), jnp.int32), hi[:-1]])
    cnt = hi - start
    pcnt = jnp.maximum((cnt + ec - 1) // ec, 1) * ec
    poff = jnp.concatenate(
        [jnp.zeros((1,), jnp.int32), jnp.cumsum(pcnt)[:-1].astype(jnp.int32)])
    p = jnp.arange(ep, dtype=jnp.int32)
    cb = jnp.clip(jnp.searchsorted(poff, p, side="right").astype(jnp.int32) - 1,
                  0, nblk - 1)
    rel = p - poff[cb]
    q = rel + start[cb]
    valid = rel < cnt[cb]
    qc = jnp.clip(q, 0, e - 1)
    gsrc = jnp.where(valid, src_s[qc], 0)
    lid = jnp.where(valid, dst_s[qc] - cb * nb, -1)
    chunk_blk = cb[:: ec]
    return (gsrc, lid.reshape(nchunk, ec, 1), lid.reshape(nchunk, 1, ec),
            chunk_blk)


def _edge_kernel(blk_ref, lidc_ref, lidr_ref, gb_ref, a_ref, o0_ref,
                 w2_ref, b2_ref, out_ref, *, nb, ec, hd):
    c = pl.program_id(0)
    cur = blk_ref[c]
    prev = blk_ref[jnp.maximum(c - 1, 0)]

    @pl.when((c == 0) | (cur != prev))
    def _init():
        out_ref[...] = o0_ref[...]

    lid_c = lidc_ref[0]
    lid_r = lidr_ref[0]
    iota_row = lax.broadcasted_iota(jnp.int32, (1, nb), 1)
    expand = (lid_c == iota_row).astype(jnp.float32)
    ae = jnp.dot(expand, a_ref[...], preferred_element_type=jnp.float32)
    s = jnp.maximum(ae + gb_ref[...], 0.0)
    m = jnp.dot(s, w2_ref[...], preferred_element_type=jnp.float32)

    k = 1
    while k < ec:
        m_sh = jnp.concatenate(
            [jnp.full((k, hd), -3e38, jnp.float32), m[:-k]], axis=0)
        l_sh = jnp.concatenate(
            [jnp.full((k, 1), -2, jnp.int32), lid_c[:-k]], axis=0)
        m = jnp.where(l_sh == lid_c, jnp.maximum(m, m_sh), m)
        k *= 2

    nxt = jnp.concatenate(
        [lid_r[:, 1:], jnp.full((1, 1), -3, jnp.int32)], axis=1)
    is_last = (lid_r != nxt) & (lid_r >= 0)
    iota_col = lax.broadcasted_iota(jnp.int32, (nb, 1), 0)
    ex = ((iota_col == lid_r) & is_last).astype(jnp.float32)
    res = jnp.dot(ex, m, preferred_element_type=jnp.float32)
    cntv = jnp.sum(ex, axis=1, keepdims=True)
    cand = jnp.where(cntv > 0, res + b2_ref[...], -3e38)
    out_ref[...] = jnp.maximum(out_ref[...], cand)


def _edge_call(chunk_blk, lid_col, lid_row, gb, a, out0, w2, b2, *,
               nb, ec, npad, hd, interpret=False):
    nchunk = chunk_blk.shape[0]
    grid_spec = pltpu.PrefetchScalarGridSpec(
        num_scalar_prefetch=1,
        grid=(nchunk,),
        in_specs=[
            pl.BlockSpec((1, ec, 1), lambda c, blk: (c, 0, 0)),
            pl.BlockSpec((1, 1, ec), lambda c, blk: (c, 0, 0)),
            pl.BlockSpec((ec, hd), lambda c, blk: (c, 0)),
            pl.BlockSpec((nb, hd), lambda c, blk: (blk[c], 0)),
            pl.BlockSpec((nb, hd), lambda c, blk: (blk[c], 0)),
            pl.BlockSpec((hd, hd), lambda c, blk: (0, 0)),
            pl.BlockSpec((1, hd), lambda c, blk: (0, 0)),
        ],
        out_specs=pl.BlockSpec((nb, hd), lambda c, blk: (blk[c], 0)),
    )
    body = functools.partial(_edge_kernel, nb=nb, ec=ec, hd=hd)
    return pl.pallas_call(
        body,
        grid_spec=grid_spec,
        out_shape=jax.ShapeDtypeStruct((npad, hd), jnp.float32),
        interpret=interpret,
    )(chunk_blk, lid_col, lid_row, gb, a, out0, w2, b2.reshape(1, hd))


def _edge_conv(h, W1, b1, W2, b2, gsrc, lid_col, lid_row, chunk_blk, *,
               nb, ec, npad, hd, interpret=False):
    W1a, W1b = W1[:hd], W1[hd:]
    A, B, out0 = _dense_phase(h, W1a - W1b, W1b, W2, b1, b2, npad=npad)
    gb = jnp.take(B, gsrc, axis=0)
    return _edge_call(chunk_blk, lid_col, lid_row, gb, A, out0, W2, b2,
                      nb=nb, ec=ec, npad=npad, hd=hd, interpret=interpret)


def kernel(x, tproj_W, tproj_b, enc_emb_W, enc_emb_b, enc_W1, enc_b1,
           enc_W2, enc_b2, enc_fc_W, enc_fc_b, dec_W1, dec_b1, dec_W2,
           dec_b2, dec_fc_W, dec_fc_b, edge_index):
    n = x.shape[0]
    timesteps = 1000
    s = 0.008
    steps = jnp.linspace(0.0, float(timesteps), timesteps + 1)
    alpha_bar = jnp.cos((steps / timesteps + s) / (1 + s) * jnp.pi / 2) ** 2
    rk = jax.random.key(42)
    k1, k2 = jax.random.split(rk)
    t = jax.random.randint(k1, (1,), 0, timesteps)
    noise = jax.random.normal(k2, x.shape, dtype=x.dtype)
    ab_t = alpha_bar[t][:, None]
    x_noisy = jnp.sqrt(ab_t) * x + jnp.sqrt(1.0 - ab_t) * noise
    tf = t.astype(jnp.float32) / timesteps
    freq = jnp.exp(jnp.linspace(-4.0, 4.0, 32))
    emb = jnp.concatenate([jnp.sin(tf * freq), jnp.cos(tf * freq)], axis=-1)
    t_emb = emb @ tproj_W + tproj_b
    h = x_noisy + t_emb[None, :]

    src, dst = edge_index[0], edge_index[1]
    gsrc, lid_col, lid_row, chunk_blk = _setup_edges(
        src, dst, nb=_NB, ec=_EC, nblk=_NBLK, ep=_EP, e=_E)
    ec_kw = dict(nb=_NB, ec=_EC, npad=_NPAD, hd=_H)

    h = jnp.pad(h, ((0, _NPAD - n), (0, 0)))
    h = _mm(h, enc_emb_W, enc_emb_b)
    for i in range(4):
        h = _edge_conv(h, enc_W1[i], enc_b1[i], enc_W2[i], enc_b2[i],
                       gsrc, lid_col, lid_row, chunk_blk, **ec_kw)
    h = _mm(h, enc_fc_W, enc_fc_b)
    for i in range(4):
        h = _edge_conv(h, dec_W1[i], dec_b1[i], dec_W2[i], dec_b2[i],
                       gsrc, lid_col, lid_row, chunk_blk, **ec_kw)
    score = _mm(h[:n], dec_fc_W, dec_fc_b)
    loss = jnp.mean((score - noise) ** 2)
    return loss
